# initial kernel scaffold (unmeasured)
import jax
import jax.numpy as jnp
from jax import lax
from jax.experimental import pallas as pl
from jax.experimental.pallas import tpu as pltpu


def kernel(
    x,
):
    def body(*refs):
        pass

    out_shape = jax.ShapeDtypeStruct(..., jnp.float32)
    return pl.pallas_call(body, out_shape=out_shape)(...)


# baseline (device time: 705190 ns/iter reference)
import jax
import jax.numpy as jnp
from jax import lax
from jax.experimental import pallas as pl
from jax.experimental.pallas import tpu as pltpu

N_DEV = 8
M, N = 4096, 2048
CH = M // N_DEV


def _ring_coords(pos):
    g = pos ^ (pos >> 1)
    return ((g >> 2) & 1, (g >> 1) & 1, g & 1)


def kernel(x):
    x = x.reshape(M, N)

    def body(x_ref, out_ref, comm_ref,
             rs_send_sems, rs_recv_sems, ag_send_sems, ag_recv_sems,
             rs_credit, ag_credit, load_sem):
        mx = lax.axis_index("x")
        my = lax.axis_index("y")
        mz = lax.axis_index("z")
        k = 4 * mx + 2 * (mx ^ my) + (mx ^ my ^ mz)
        right = _ring_coords((k + 1) % N_DEV)
        left = _ring_coords((k + N_DEV - 1) % N_DEV)

        load = pltpu.make_async_copy(x_ref, out_ref, load_sem)
        load.start()

        barrier_sem = pltpu.get_barrier_semaphore()
        for nbr in (left, right):
            pl.semaphore_signal(
                barrier_sem, inc=1,
                device_id=nbr, device_id_type=pl.DeviceIdType.MESH,
            )
        pl.semaphore_wait(barrier_sem, 2)
        load.wait()

        for s in range(N_DEV - 1):
            slot = s % 2
            if s >= 2:
                pl.semaphore_wait(rs_credit, 1)
            send_chunk = (k - s) % N_DEV
            recv_chunk = (k - s - 1) % N_DEV
            rdma = pltpu.make_async_remote_copy(
                src_ref=out_ref.at[pl.ds(send_chunk * CH, CH), :],
                dst_ref=comm_ref.at[slot],
                send_sem=rs_send_sems.at[slot],
                recv_sem=rs_recv_sems.at[slot],
                device_id=right,
                device_id_type=pl.DeviceIdType.MESH,
            )
            rdma.start()
            rdma.wait()
            rows = pl.ds(recv_chunk * CH, CH)
            out_ref[rows, :] = out_ref[rows, :] + comm_ref[slot, :, :]
            pl.semaphore_signal(
                rs_credit, inc=1,
                device_id=left, device_id_type=pl.DeviceIdType.MESH,
            )
        pl.semaphore_wait(rs_credit, 2)

        for s in range(N_DEV - 1):
            slot = s % 2
            if s >= 2:
                pl.semaphore_wait(ag_credit, 1)
            send_chunk = (k + 1 - s) % N_DEV
            rows = pl.ds(send_chunk * CH, CH)
            rdma = pltpu.make_async_remote_copy(
                src_ref=out_ref.at[rows, :],
                dst_ref=out_ref.at[rows, :],
                send_sem=ag_send_sems.at[slot],
                recv_sem=ag_recv_sems.at[slot],
                device_id=right,
                device_id_type=pl.DeviceIdType.MESH,
            )
            rdma.start()
            rdma.wait()
            pl.semaphore_signal(
                ag_credit, inc=1,
                device_id=left, device_id_type=pl.DeviceIdType.MESH,
            )
        pl.semaphore_wait(ag_credit, 2)

    return pl.pallas_call(
        body,
        out_shape=jax.ShapeDtypeStruct((M, N), jnp.float32),
        in_specs=[pl.BlockSpec(memory_space=pl.ANY)],
        out_specs=pl.BlockSpec(memory_space=pltpu.VMEM),
        scratch_shapes=[
            pltpu.VMEM((2, CH, N), jnp.float32),
            pltpu.SemaphoreType.DMA((2,)),
            pltpu.SemaphoreType.DMA((2,)),
            pltpu.SemaphoreType.DMA((2,)),
            pltpu.SemaphoreType.DMA((2,)),
            pltpu.SemaphoreType.REGULAR,
            pltpu.SemaphoreType.REGULAR,
            pltpu.SemaphoreType.DMA,

        ],
        compiler_params=pltpu.CompilerParams(
            collective_id=0,
            vmem_limit_bytes=60 * 1024 * 1024,
        ),
    )(x)


# device time: 277299 ns/iter; 2.5431x vs baseline; 2.5431x over previous
import jax
import jax.numpy as jnp
from jax import lax
from jax.experimental import pallas as pl
from jax.experimental.pallas import tpu as pltpu

M, N = 4096, 2048

BANDS = (
    (0, 1408, ("x", "y", "z")),
    (1408, 1344, ("y", "z", "x")),
    (2752, 1344, ("z", "x", "y")),
)


def kernel(x):
    x = x.reshape(M, N)

    def body(x_ref, out_ref, comm0, comm1, comm2,
             rs_send, rs_recv, ag_send, ag_recv, credits, load_sem):
        comms = (comm0, comm1, comm2)
        coord = {
            "x": lax.axis_index("x"),
            "y": lax.axis_index("y"),
            "z": lax.axis_index("z"),
        }

        def partner(d):
            return tuple(
                1 - coord[a] if a == d else coord[a] for a in ("x", "y", "z")
            )

        load = pltpu.make_async_copy(x_ref, out_ref, load_sem)
        load.start()

        barrier_sem = pltpu.get_barrier_semaphore()
        for d in ("x", "y", "z"):
            pl.semaphore_signal(
                barrier_sem, inc=1,
                device_id=partner(d), device_id_type=pl.DeviceIdType.MESH,
            )
        pl.semaphore_wait(barrier_sem, 3)
        load.wait()

        offs = [[base] for (base, _, _) in BANDS]

        for s in range(3):
            rdmas = []
            for b, (base, R, dims) in enumerate(BANDS):
                h = R >> (s + 1)
                d = dims[s]
                bit = coord[d]
                off = offs[b][s]
                send_off = off + (1 - bit) * h
                if s >= 1:
                    pl.semaphore_wait(credits.at[b, s - 1], 1)
                rdma = pltpu.make_async_remote_copy(
                    src_ref=out_ref.at[pl.ds(send_off, h), :],
                    dst_ref=comms[b].at[pl.ds(0, h), :],
                    send_sem=rs_send.at[b, s],
                    recv_sem=rs_recv.at[b, s],
                    device_id=partner(d),
                    device_id_type=pl.DeviceIdType.MESH,
                )
                rdma.start()
                rdmas.append(rdma)
                offs[b].append(off + bit * h)
            for b, (base, R, dims) in enumerate(BANDS):
                h = R >> (s + 1)
                rdmas[b].wait()
                rows = pl.ds(offs[b][s + 1], h)
                out_ref[rows, :] = out_ref[rows, :] + comms[b][pl.ds(0, h), :]
                if s <= 1:
                    pl.semaphore_signal(
                        credits.at[b, s], inc=1,
                        device_id=partner(dims[s + 1]),
                        device_id_type=pl.DeviceIdType.MESH,
                    )

        for s in (2, 1, 0):
            rdmas = []
            for b, (base, R, dims) in enumerate(BANDS):
                h = R >> (s + 1)
                rows = pl.ds(offs[b][s + 1], h)
                rdma = pltpu.make_async_remote_copy(
                    src_ref=out_ref.at[rows, :],
                    dst_ref=out_ref.at[rows, :],
                    send_sem=ag_send.at[b, s],
                    recv_sem=ag_recv.at[b, s],
                    device_id=partner(dims[s]),
                    device_id_type=pl.DeviceIdType.MESH,
                )
                rdma.start()
                rdmas.append(rdma)
            for b in range(3):
                rdmas[b].wait()

    return pl.pallas_call(
        body,
        out_shape=jax.ShapeDtypeStruct((M, N), jnp.float32),
        in_specs=[pl.BlockSpec(memory_space=pl.ANY)],
        out_specs=pl.BlockSpec(memory_space=pltpu.VMEM),
        scratch_shapes=[
            pltpu.VMEM((BANDS[0][1] // 2, N), jnp.float32),
            pltpu.VMEM((BANDS[1][1] // 2, N), jnp.float32),
            pltpu.VMEM((BANDS[2][1] // 2, N), jnp.float32),
            pltpu.SemaphoreType.DMA((3, 3)),
            pltpu.SemaphoreType.DMA((3, 3)),
            pltpu.SemaphoreType.DMA((3, 3)),
            pltpu.SemaphoreType.DMA((3, 3)),
            pltpu.SemaphoreType.REGULAR((3, 2)),
            pltpu.SemaphoreType.DMA,
        ],
        compiler_params=pltpu.CompilerParams(
            collective_id=0,
            vmem_limit_bytes=60 * 1024 * 1024,
        ),
    )(x)


# device time: 273215 ns/iter; 2.5811x vs baseline; 1.0149x over previous
import jax
import jax.numpy as jnp
from jax import lax
from jax.experimental import pallas as pl
from jax.experimental.pallas import tpu as pltpu

M, N = 4096, 2048

BANDS = (
    (0, 1408, ("x", "y", "z")),
    (1408, 1344, ("y", "z", "x")),
    (2752, 1344, ("z", "x", "y")),
)


def kernel(x):
    x = x.reshape(M, N)

    def body(x_ref, out_ref, comm0, comm1, comm2,
             rs_send, rs_recv, ag_send, ag_recv, load_sem):
        comms = (comm0, comm1, comm2)
        coord = {
            "x": lax.axis_index("x"),
            "y": lax.axis_index("y"),
            "z": lax.axis_index("z"),
        }

        def partner(d):
            return tuple(
                1 - coord[a] if a == d else coord[a] for a in ("x", "y", "z")
            )

        load = pltpu.make_async_copy(x_ref, out_ref, load_sem)
        load.start()

        barrier_sem = pltpu.get_barrier_semaphore()
        for d in ("x", "y", "z"):
            pl.semaphore_signal(
                barrier_sem, inc=1,
                device_id=partner(d), device_id_type=pl.DeviceIdType.MESH,
            )
        pl.semaphore_wait(barrier_sem, 3)
        load.wait()

        offs, send_offs, parts = [], [], []
        for base, R, dims in BANDS:
            o = [base]
            so = []
            pt = []
            for s in range(3):
                h = R >> (s + 1)
                bit = coord[dims[s]]
                so.append(o[s] + (1 - bit) * h)
                o.append(o[s] + bit * h)
                pt.append(partner(dims[s]))
            offs.append(o)
            send_offs.append(so)
            parts.append(pt)
        cums = [(0, R >> 1, (R >> 1) + (R >> 2)) for (_, R, _) in BANDS]

        def rs_rdma(b, s):
            h = BANDS[b][1] >> (s + 1)
            return pltpu.make_async_remote_copy(
                src_ref=out_ref.at[pl.ds(send_offs[b][s], h), :],
                dst_ref=comms[b].at[pl.ds(cums[b][s], h), :],
                send_sem=rs_send.at[b, s],
                recv_sem=rs_recv.at[b, s],
                device_id=parts[b][s],
                device_id_type=pl.DeviceIdType.MESH,
            )

        def ag_rdma(b, s):
            h = BANDS[b][1] >> (s + 1)
            rows = pl.ds(offs[b][s + 1], h)
            return pltpu.make_async_remote_copy(
                src_ref=out_ref.at[rows, :],
                dst_ref=out_ref.at[rows, :],
                send_sem=ag_send.at[b, s],
                recv_sem=ag_recv.at[b, s],
                device_id=parts[b][s],
                device_id_type=pl.DeviceIdType.MESH,
            )

        rdmas = []
        for b in range(3):
            r = rs_rdma(b, 0)
            r.start()
            rdmas.append(r)
        ag_rdmas = [None, None, None]
        for s in range(3):
            for b in range(3):
                rdmas[b].wait()
                h = BANDS[b][1] >> (s + 1)
                hn = h >> 1
                kept = offs[b][s + 1]
                cbase = cums[b][s]
                if s < 2:
                    j = send_offs[b][s + 1] - kept
                    rows = pl.ds(send_offs[b][s + 1], hn)
                    out_ref[rows, :] = (
                        out_ref[rows, :] + comms[b][pl.ds(cbase + j, hn), :]
                    )
                    nxt = rs_rdma(b, s + 1)
                    nxt.start()
                    rdmas[b] = nxt
                    jk = offs[b][s + 2] - kept
                    rows = pl.ds(offs[b][s + 2], hn)
                    out_ref[rows, :] = (
                        out_ref[rows, :] + comms[b][pl.ds(cbase + jk, hn), :]
                    )
                else:
                    rows = pl.ds(offs[b][3], h)
                    out_ref[rows, :] = (
                        out_ref[rows, :] + comms[b][pl.ds(cbase, h), :]
                    )
                    r = ag_rdma(b, 2)
                    r.start()
                    ag_rdmas[b] = r

        for s in (2, 1, 0):
            for b in range(3):
                ag_rdmas[b].wait()
                if s > 0:
                    nxt = ag_rdma(b, s - 1)
                    nxt.start()
                    ag_rdmas[b] = nxt

    return pl.pallas_call(
        body,
        out_shape=jax.ShapeDtypeStruct((M, N), jnp.float32),
        in_specs=[pl.BlockSpec(memory_space=pl.ANY)],
        out_specs=pl.BlockSpec(memory_space=pltpu.VMEM),
        scratch_shapes=[
            pltpu.VMEM((BANDS[0][1] * 7 // 8, N), jnp.float32),
            pltpu.VMEM((BANDS[1][1] * 7 // 8, N), jnp.float32),
            pltpu.VMEM((BANDS[2][1] * 7 // 8, N), jnp.float32),
            pltpu.SemaphoreType.DMA((3, 3)),
            pltpu.SemaphoreType.DMA((3, 3)),
            pltpu.SemaphoreType.DMA((3, 3)),
            pltpu.SemaphoreType.DMA((3, 3)),
            pltpu.SemaphoreType.DMA,
        ],
        compiler_params=pltpu.CompilerParams(
            collective_id=0,
            vmem_limit_bytes=63 * 1024 * 1024,
        ),
    )(x)


# device time: 261664 ns/iter; 2.6950x vs baseline; 1.0441x over previous
import jax
import jax.numpy as jnp
from jax import lax
from jax.experimental import pallas as pl
from jax.experimental.pallas import tpu as pltpu

M, N = 4096, 2048
CC = N // 2

BANDS = (
    (0, 1408, ("x", "y", "z")),
    (1408, 1344, ("y", "z", "x")),
    (2752, 1344, ("z", "x", "y")),
)


def kernel(x):
    x = x.reshape(M, N)

    def body(x_ref, out_ref, comm0, comm1, comm2,
             rs_send, rs_recv, ag_send, ag_recv, load_sems):
        comms = (comm0, comm1, comm2)
        coord = {
            "x": lax.axis_index("x"),
            "y": lax.axis_index("y"),
            "z": lax.axis_index("z"),
        }

        def partner(d):
            return tuple(
                1 - coord[a] if a == d else coord[a] for a in ("x", "y", "z")
            )

        offs, send_offs, parts = [], [], []
        for base, R, dims in BANDS:
            o = [base]
            so = []
            pt = []
            for s in range(3):
                h = R >> (s + 1)
                bit = coord[dims[s]]
                so.append(o[s] + (1 - bit) * h)
                o.append(o[s] + bit * h)
                pt.append(partner(dims[s]))
            offs.append(o)
            send_offs.append(so)
            parts.append(pt)
        cums = [(0, R >> 1, (R >> 1) + (R >> 2)) for (_, R, _) in BANDS]

        loads = []
        for b, (base, R, dims) in enumerate(BANDS):
            h0 = R >> 1
            pair = []
            for t, roff in enumerate((send_offs[b][0], offs[b][1])):
                rows = pl.ds(roff, h0)
                ld = pltpu.make_async_copy(
                    x_ref.at[rows, :], out_ref.at[rows, :],
                    load_sems.at[b, t],
                )
                ld.start()
                pair.append(ld)
            loads.append(pair)

        barrier_sem = pltpu.get_barrier_semaphore()
        for d in ("x", "y", "z"):
            pl.semaphore_signal(
                barrier_sem, inc=1,
                device_id=partner(d), device_id_type=pl.DeviceIdType.MESH,
            )
        pl.semaphore_wait(barrier_sem, 3)

        def rs_rdma(b, s, c):
            h = BANDS[b][1] >> (s + 1)
            cols = pl.ds(c * CC, CC)
            return pltpu.make_async_remote_copy(
                src_ref=out_ref.at[pl.ds(send_offs[b][s], h), cols],
                dst_ref=comms[b].at[pl.ds(cums[b][s], h), cols],
                send_sem=rs_send.at[b, s, c],
                recv_sem=rs_recv.at[b, s, c],
                device_id=parts[b][s],
                device_id_type=pl.DeviceIdType.MESH,
            )

        def ag_rdma(b, s, c):
            h = BANDS[b][1] >> (s + 1)
            rows = pl.ds(offs[b][s + 1], h)
            cols = pl.ds(c * CC, CC)
            return pltpu.make_async_remote_copy(
                src_ref=out_ref.at[rows, cols],
                dst_ref=out_ref.at[rows, cols],
                send_sem=ag_send.at[b, s, c],
                recv_sem=ag_recv.at[b, s, c],
                device_id=parts[b][s],
                device_id_type=pl.DeviceIdType.MESH,
            )

        rdmas = [[None, None] for _ in range(3)]
        for b in range(3):
            loads[b][0].wait()
            for c in (0, 1):
                r = rs_rdma(b, 0, c)
                r.start()
                rdmas[b][c] = r
        ag_rdmas = [[None, None] for _ in range(3)]
        for s in range(3):
            for b in range(3):
                for c in (0, 1):
                    rdmas[b][c].wait()
                    if s == 0 and c == 0:
                        loads[b][1].wait()
                    h = BANDS[b][1] >> (s + 1)
                    hn = h >> 1
                    kept = offs[b][s + 1]
                    cbase = cums[b][s]
                    cols = pl.ds(c * CC, CC)
                    if s < 2:
                        j = send_offs[b][s + 1] - kept
                        rows = pl.ds(send_offs[b][s + 1], hn)
                        out_ref[rows, cols] = (
                            out_ref[rows, cols]
                            + comms[b][pl.ds(cbase + j, hn), cols]
                        )
                        nxt = rs_rdma(b, s + 1, c)
                        nxt.start()
                        rdmas[b][c] = nxt
                        jk = offs[b][s + 2] - kept
                        rows = pl.ds(offs[b][s + 2], hn)
                        out_ref[rows, cols] = (
                            out_ref[rows, cols]
                            + comms[b][pl.ds(cbase + jk, hn), cols]
                        )
                    else:
                        rows = pl.ds(offs[b][3], h)
                        out_ref[rows, cols] = (
                            out_ref[rows, cols]
                            + comms[b][pl.ds(cbase, h), cols]
                        )
                        r = ag_rdma(b, 2, c)
                        r.start()
                        ag_rdmas[b][c] = r

        for s in (2, 1, 0):
            for b in range(3):
                for c in (0, 1):
                    ag_rdmas[b][c].wait()
                    if s > 0:
                        nxt = ag_rdma(b, s - 1, c)
                        nxt.start()
                        ag_rdmas[b][c] = nxt

    return pl.pallas_call(
        body,
        out_shape=jax.ShapeDtypeStruct((M, N), jnp.float32),
        in_specs=[pl.BlockSpec(memory_space=pl.ANY)],
        out_specs=pl.BlockSpec(memory_space=pltpu.VMEM),
        scratch_shapes=[
            pltpu.VMEM((BANDS[0][1] * 7 // 8, N), jnp.float32),
            pltpu.VMEM((BANDS[1][1] * 7 // 8, N), jnp.float32),
            pltpu.VMEM((BANDS[2][1] * 7 // 8, N), jnp.float32),
            pltpu.SemaphoreType.DMA((3, 3, 2)),
            pltpu.SemaphoreType.DMA((3, 3, 2)),
            pltpu.SemaphoreType.DMA((3, 3, 2)),
            pltpu.SemaphoreType.DMA((3, 3, 2)),
            pltpu.SemaphoreType.DMA((3, 2)),
        ],
        compiler_params=pltpu.CompilerParams(
            collective_id=0,
            vmem_limit_bytes=63 * 1024 * 1024,
        ),
    )(x)


# device time: 256538 ns/iter; 2.7489x vs baseline; 1.0200x over previous
import jax
import jax.numpy as jnp
from jax import lax
from jax.experimental import pallas as pl
from jax.experimental.pallas import tpu as pltpu

M, N = 4096, 2048
CHUNKS = 4
CC = N // CHUNKS

BANDS = (
    (0, 1408, ("x", "y", "z")),
    (1408, 1344, ("y", "z", "x")),
    (2752, 1344, ("z", "x", "y")),
)


def kernel(x):
    x = x.reshape(M, N)

    def body(x_ref, out_ref, comm0, comm1, comm2,
             rs_send, rs_recv, ag_send, ag_recv, load_sems):
        comms = (comm0, comm1, comm2)
        coord = {
            "x": lax.axis_index("x"),
            "y": lax.axis_index("y"),
            "z": lax.axis_index("z"),
        }

        def partner(d):
            return tuple(
                1 - coord[a] if a == d else coord[a] for a in ("x", "y", "z")
            )

        offs, send_offs, parts = [], [], []
        for base, R, dims in BANDS:
            o = [base]
            so = []
            pt = []
            for s in range(3):
                h = R >> (s + 1)
                bit = coord[dims[s]]
                so.append(o[s] + (1 - bit) * h)
                o.append(o[s] + bit * h)
                pt.append(partner(dims[s]))
            offs.append(o)
            send_offs.append(so)
            parts.append(pt)
        cums = [(0, R >> 1, (R >> 1) + (R >> 2)) for (_, R, _) in BANDS]

        loads = []
        for b, (base, R, dims) in enumerate(BANDS):
            h0 = R >> 1
            pair = []
            for t, roff in enumerate((send_offs[b][0], offs[b][1])):
                rows = pl.ds(roff, h0)
                ld = pltpu.make_async_copy(
                    x_ref.at[rows, :], out_ref.at[rows, :],
                    load_sems.at[b, t],
                )
                ld.start()
                pair.append(ld)
            loads.append(pair)

        barrier_sem = pltpu.get_barrier_semaphore()
        for d in ("x", "y", "z"):
            pl.semaphore_signal(
                barrier_sem, inc=1,
                device_id=partner(d), device_id_type=pl.DeviceIdType.MESH,
            )
        pl.semaphore_wait(barrier_sem, 3)

        def rs_rdma(b, s, c):
            h = BANDS[b][1] >> (s + 1)
            cols = pl.ds(c * CC, CC)
            return pltpu.make_async_remote_copy(
                src_ref=out_ref.at[pl.ds(send_offs[b][s], h), cols],
                dst_ref=comms[b].at[pl.ds(cums[b][s], h), cols],
                send_sem=rs_send.at[b, s, c],
                recv_sem=rs_recv.at[b, s, c],
                device_id=parts[b][s],
                device_id_type=pl.DeviceIdType.MESH,
            )

        def ag_rdma(b, s, c):
            h = BANDS[b][1] >> (s + 1)
            rows = pl.ds(offs[b][s + 1], h)
            cols = pl.ds(c * CC, CC)
            return pltpu.make_async_remote_copy(
                src_ref=out_ref.at[rows, cols],
                dst_ref=out_ref.at[rows, cols],
                send_sem=ag_send.at[b, s, c],
                recv_sem=ag_recv.at[b, s, c],
                device_id=parts[b][s],
                device_id_type=pl.DeviceIdType.MESH,
            )

        rdmas = [[None] * CHUNKS for _ in range(3)]
        for b in range(3):
            loads[b][0].wait()
            for c in range(CHUNKS):
                r = rs_rdma(b, 0, c)
                r.start()
                rdmas[b][c] = r
        ag_rdmas = [[None] * CHUNKS for _ in range(3)]
        for s in range(3):
            for c in range(CHUNKS):
                for b in range(3):
                    rdmas[b][c].wait()
                    if s == 0 and c == 0:
                        loads[b][1].wait()
                    h = BANDS[b][1] >> (s + 1)
                    hn = h >> 1
                    kept = offs[b][s + 1]
                    cbase = cums[b][s]
                    cols = pl.ds(c * CC, CC)
                    if s < 2:
                        j = send_offs[b][s + 1] - kept
                        rows = pl.ds(send_offs[b][s + 1], hn)
                        out_ref[rows, cols] = (
                            out_ref[rows, cols]
                            + comms[b][pl.ds(cbase + j, hn), cols]
                        )
                        nxt = rs_rdma(b, s + 1, c)
                        nxt.start()
                        rdmas[b][c] = nxt
                        jk = offs[b][s + 2] - kept
                        rows = pl.ds(offs[b][s + 2], hn)
                        out_ref[rows, cols] = (
                            out_ref[rows, cols]
                            + comms[b][pl.ds(cbase + jk, hn), cols]
                        )
                    else:
                        rows = pl.ds(offs[b][3], h)
                        out_ref[rows, cols] = (
                            out_ref[rows, cols]
                            + comms[b][pl.ds(cbase, h), cols]
                        )
                        r = ag_rdma(b, 2, c)
                        r.start()
                        ag_rdmas[b][c] = r

        for s in (2, 1, 0):
            for c in range(CHUNKS):
                for b in range(3):
                    ag_rdmas[b][c].wait()
                    if s > 0:
                        nxt = ag_rdma(b, s - 1, c)
                        nxt.start()
                        ag_rdmas[b][c] = nxt

    return pl.pallas_call(
        body,
        out_shape=jax.ShapeDtypeStruct((M, N), jnp.float32),
        in_specs=[pl.BlockSpec(memory_space=pl.ANY)],
        out_specs=pl.BlockSpec(memory_space=pltpu.VMEM),
        scratch_shapes=[
            pltpu.VMEM((BANDS[0][1] * 7 // 8, N), jnp.float32),
            pltpu.VMEM((BANDS[1][1] * 7 // 8, N), jnp.float32),
            pltpu.VMEM((BANDS[2][1] * 7 // 8, N), jnp.float32),
            pltpu.SemaphoreType.DMA((3, 3, CHUNKS)),
            pltpu.SemaphoreType.DMA((3, 3, CHUNKS)),
            pltpu.SemaphoreType.DMA((3, 3, CHUNKS)),
            pltpu.SemaphoreType.DMA((3, 3, CHUNKS)),
            pltpu.SemaphoreType.DMA((3, 2)),
        ],
        compiler_params=pltpu.CompilerParams(
            collective_id=0,
            vmem_limit_bytes=63 * 1024 * 1024,
        ),
    )(x)


# device time: 253716 ns/iter; 2.7794x vs baseline; 1.0111x over previous
import jax
import jax.numpy as jnp
from jax import lax
from jax.experimental import pallas as pl
from jax.experimental.pallas import tpu as pltpu

M, N = 4096, 2048
CHUNKS = 4
CC = N // CHUNKS

BANDS = (
    (0, 1408, ("x", "y", "z")),
    (1408, 1344, ("y", "z", "x")),
    (2752, 1344, ("z", "x", "y")),
)


def kernel(x):
    x = x.reshape(M, N)

    def body(x_ref, out_ref, comm0, comm1, comm2,
             rs_send, rs_recv, ag_send, ag_recv, load_sems):
        comms = (comm0, comm1, comm2)
        coord = {
            "x": lax.axis_index("x"),
            "y": lax.axis_index("y"),
            "z": lax.axis_index("z"),
        }

        def partner(d):
            return tuple(
                1 - coord[a] if a == d else coord[a] for a in ("x", "y", "z")
            )

        offs, send_offs, parts = [], [], []
        for base, R, dims in BANDS:
            o = [base]
            so = []
            pt = []
            for s in range(3):
                h = R >> (s + 1)
                bit = coord[dims[s]]
                so.append(o[s] + (1 - bit) * h)
                o.append(o[s] + bit * h)
                pt.append(partner(dims[s]))
            offs.append(o)
            send_offs.append(so)
            parts.append(pt)
        cums = [(0, R >> 1, (R >> 1) + (R >> 2)) for (_, R, _) in BANDS]

        loads = []
        for b, (base, R, dims) in enumerate(BANDS):
            rows = pl.ds(offs[b][1], R >> 1)
            ld = pltpu.make_async_copy(
                x_ref.at[rows, :], out_ref.at[rows, :], load_sems.at[b],
            )
            ld.start()
            loads.append(ld)

        barrier_sem = pltpu.get_barrier_semaphore()
        for d in ("x", "y", "z"):
            pl.semaphore_signal(
                barrier_sem, inc=1,
                device_id=partner(d), device_id_type=pl.DeviceIdType.MESH,
            )
        pl.semaphore_wait(barrier_sem, 3)

        def rs_rdma(b, s, c):
            h = BANDS[b][1] >> (s + 1)
            cols = pl.ds(c * CC, CC)
            src = x_ref if s == 0 else out_ref
            return pltpu.make_async_remote_copy(
                src_ref=src.at[pl.ds(send_offs[b][s], h), cols],
                dst_ref=comms[b].at[pl.ds(cums[b][s], h), cols],
                send_sem=rs_send.at[b, s, c],
                recv_sem=rs_recv.at[b, s, c],
                device_id=parts[b][s],
                device_id_type=pl.DeviceIdType.MESH,
            )

        def ag_rdma(b, s, c):
            h = BANDS[b][1] >> (s + 1)
            rows = pl.ds(offs[b][s + 1], h)
            cols = pl.ds(c * CC, CC)
            return pltpu.make_async_remote_copy(
                src_ref=out_ref.at[rows, cols],
                dst_ref=out_ref.at[rows, cols],
                send_sem=ag_send.at[b, s, c],
                recv_sem=ag_recv.at[b, s, c],
                device_id=parts[b][s],
                device_id_type=pl.DeviceIdType.MESH,
            )

        rdmas = [[None] * CHUNKS for _ in range(3)]
        for b in range(3):
            for c in range(CHUNKS):
                r = rs_rdma(b, 0, c)
                r.start()
                rdmas[b][c] = r
        ag_rdmas = [[None] * CHUNKS for _ in range(3)]
        for s in range(3):
            for c in range(CHUNKS):
                for b in range(3):
                    rdmas[b][c].wait()
                    if s == 0 and c == 0:
                        loads[b].wait()
                    h = BANDS[b][1] >> (s + 1)
                    hn = h >> 1
                    kept = offs[b][s + 1]
                    cbase = cums[b][s]
                    cols = pl.ds(c * CC, CC)
                    if s < 2:
                        j = send_offs[b][s + 1] - kept
                        rows = pl.ds(send_offs[b][s + 1], hn)
                        out_ref[rows, cols] = (
                            out_ref[rows, cols]
                            + comms[b][pl.ds(cbase + j, hn), cols]
                        )
                        nxt = rs_rdma(b, s + 1, c)
                        nxt.start()
                        rdmas[b][c] = nxt
                        jk = offs[b][s + 2] - kept
                        rows = pl.ds(offs[b][s + 2], hn)
                        out_ref[rows, cols] = (
                            out_ref[rows, cols]
                            + comms[b][pl.ds(cbase + jk, hn), cols]
                        )
                    else:
                        rows = pl.ds(offs[b][3], h)
                        out_ref[rows, cols] = (
                            out_ref[rows, cols]
                            + comms[b][pl.ds(cbase, h), cols]
                        )
                        r = ag_rdma(b, 2, c)
                        r.start()
                        ag_rdmas[b][c] = r

        for s in (2, 1, 0):
            for c in range(CHUNKS):
                for b in range(3):
                    ag_rdmas[b][c].wait()
                    if s > 0:
                        nxt = ag_rdma(b, s - 1, c)
                        nxt.start()
                        ag_rdmas[b][c] = nxt

    return pl.pallas_call(
        body,
        out_shape=jax.ShapeDtypeStruct((M, N), jnp.float32),
        in_specs=[pl.BlockSpec(memory_space=pl.ANY)],
        out_specs=pl.BlockSpec(memory_space=pltpu.VMEM),
        scratch_shapes=[
            pltpu.VMEM((BANDS[0][1] * 7 // 8, N), jnp.float32),
            pltpu.VMEM((BANDS[1][1] * 7 // 8, N), jnp.float32),
            pltpu.VMEM((BANDS[2][1] * 7 // 8, N), jnp.float32),
            pltpu.SemaphoreType.DMA((3, 3, CHUNKS)),
            pltpu.SemaphoreType.DMA((3, 3, CHUNKS)),
            pltpu.SemaphoreType.DMA((3, 3, CHUNKS)),
            pltpu.SemaphoreType.DMA((3, 3, CHUNKS)),
            pltpu.SemaphoreType.DMA((3,)),
        ],
        compiler_params=pltpu.CompilerParams(
            collective_id=0,
            vmem_limit_bytes=63 * 1024 * 1024,
        ),
    )(x)
